# baseline (speedup 1.0000x reference)
import functools

import jax
import jax.numpy as jnp
from jax import lax
from jax.experimental import pallas as pl
from jax.experimental.pallas import tpu as pltpu
from jax.experimental.pallas import tpu_sc as plsc

N = 10000
E = 320000
D = 128
H = 128
C = 10
G = 64

NC = 2
NS = 16
NW = NC * NS
EPT = E // NW
K = 128
CH = 80
EPAD = CH * K
GS = 4
NG = CH // GS
ROWS_PER_TILE = 632
AGG_ROWS = ROWS_PER_TILE * NS


def _sc_agg(h, pidx, zeros):
    mesh = plsc.VectorSubcoreMesh(core_axis_name="c", subcore_axis_name="s")

    @functools.partial(
        pl.kernel,
        mesh=mesh,
        out_type=jax.ShapeDtypeStruct((NC, AGG_ROWS, H), jnp.float32),
        scratch_types=[
            pltpu.VMEM((GS, 2, K), jnp.int32),
            pltpu.VMEM((GS, 2, K), jnp.int32),
            pltpu.VMEM((GS, 2, K), jnp.int32),
            pltpu.VMEM((K, H), jnp.float32),
            pltpu.VMEM((K, H), jnp.float32),
            pltpu.VMEM_SHARED((AGG_ROWS, H), jnp.float32),
            pltpu.SemaphoreType.DMA,
            pltpu.SemaphoreType.DMA,
            pltpu.SemaphoreType.DMA,
            pltpu.SemaphoreType.DMA,
            pltpu.SemaphoreType.DMA,
            pltpu.SemaphoreType.DMA,
            pltpu.SemaphoreType.DMA,
        ],
    )
    def k(h_hbm, pidx_hbm, z_hbm, out_hbm,
          ibuf0, ibuf1, ibuf2, rows0, rows1, agg_sh,
          isem0, isem1, isem2, gsem0, gsem1, ssem0, ssem1):
        c = lax.axis_index("c")
        s = lax.axis_index("s")
        wid = c * NS + s
        ibuf = (ibuf0, ibuf1, ibuf2)
        rows = (rows0, rows1)
        isem = (isem0, isem1, isem2)
        gsem = (gsem0, gsem1)
        ssem = (ssem0, ssem1)

        pltpu.sync_copy(z_hbm.at[pl.ds(s * ROWS_PER_TILE, ROWS_PER_TILE)],
                        agg_sh.at[pl.ds(s * ROWS_PER_TILE, ROWS_PER_TILE)])
        plsc.subcore_barrier()

        def i_start(g, gb):
            pltpu.async_copy(pidx_hbm.at[wid, g], ibuf[gb], isem[gb])

        def i_wait(gb):
            pltpu.make_async_copy(pidx_hbm.at[0, 0], ibuf[gb],
                                  isem[gb]).wait()

        def g_start(t, gb, b):
            pltpu.async_copy(h_hbm.at[ibuf[gb].at[t, 0]], rows[b], gsem[b])

        def g_wait(b):
            pltpu.make_async_copy(h_hbm.at[pl.ds(0, K)], rows[b],
                                  gsem[b]).wait()

        def s_start(t, gb, b):
            pltpu.async_copy(rows[b], agg_sh.at[ibuf[gb].at[t, 1]], ssem[b],
                             add=True)

        def s_wait(b):
            pltpu.make_async_copy(rows[b], agg_sh.at[pl.ds(0, K)],
                                  ssem[b]).wait()

        i_start(0, 0)
        i_start(1, 1)
        i_wait(0)
        g_start(0, 0, 0)
        g_start(1, 0, 1)
        g_wait(0)
        s_start(0, 0, 0)
        s_wait(0)
        g_start(2, 0, 0)
        g_wait(1)
        s_start(1, 0, 1)
        i_start(2, 2)
        s_wait(1)
        g_start(3, 0, 1)
        g_wait(0)
        s_start(2, 0, 0)
        i_wait(1)
        s_wait(0)
        g_start(0, 1, 0)
        g_wait(1)
        s_start(3, 0, 1)
        s_wait(1)
        g_start(1, 1, 1)
        g_wait(0)
        s_start(0, 1, 0)
        s_wait(0)
        g_start(2, 1, 0)
        g_wait(1)
        s_start(1, 1, 1)
        i_start(3, 0)
        s_wait(1)
        g_start(3, 1, 1)
        g_wait(0)
        s_start(2, 1, 0)

        def block(jj, carry):
            for cst in range(3):
                g = 3 * jj + 2 + cst
                gb = (2 + cst) % 3
                pgb = (1 + cst) % 3
                for t in range(GS):
                    b = t % 2
                    pt, p_gb = (GS - 1, pgb) if t == 0 else (t - 1, gb)
                    if t == 0:
                        i_wait(gb)
                    s_wait(b)
                    g_start(t, gb, b)
                    g_wait(1 - b)
                    s_start(pt, p_gb, 1 - b)
                    if t == 2:
                        @pl.when(g + 2 < NG)
                        def _():
                            i_start(g + 2, (1 + cst) % 3)
            return carry

        lax.fori_loop(0, (NG - 2) // 3, block, 0)

        g_wait(1)
        s_start(GS - 1, 1, 1)
        s_wait(0)
        s_wait(1)

        plsc.subcore_barrier()
        pltpu.sync_copy(agg_sh.at[pl.ds(s * ROWS_PER_TILE, ROWS_PER_TILE)],
                        out_hbm.at[c, pl.ds(s * ROWS_PER_TILE, ROWS_PER_TILE)])

    return k(h, pidx, zeros)


def _tc_layer(h, agg2, scale, W1, b1, W2, b2, gamma, beta):
    def body(h_ref, agg_ref, sc_ref, w1_ref, b1_ref, w2_ref, b2_ref,
             g_ref, be_ref, out_ref):
        z = (h_ref[...] * sc_ref[0, 0] + agg_ref[0, pl.ds(0, N)]
             + agg_ref[1, pl.ds(0, N)])
        z = jnp.maximum(
            jnp.dot(z, w1_ref[...], preferred_element_type=jnp.float32) + b1_ref[...], 0.0)
        z = jnp.maximum(
            jnp.dot(z, w2_ref[...], preferred_element_type=jnp.float32) + b2_ref[...], 0.0)
        mean = jnp.sum(z, axis=0, keepdims=True) * (1.0 / N)
        d = z - mean
        var = jnp.sum(d * d, axis=0, keepdims=True) * (1.0 / N)
        zn = d * (g_ref[...] * lax.rsqrt(var + 1e-5)) + be_ref[...]
        out_ref[...] = jnp.maximum(zn, 0.0)

    return pl.pallas_call(
        body,
        out_shape=jax.ShapeDtypeStruct((N, H), jnp.float32),
    )(h, agg2, scale, W1, b1, W2, b2, gamma, beta)


def _tc_head(h, batch2d, fc1W, fc1b, fc2W, fc2b, fc3W, fc3b):
    def body(h_ref, b_ref, w1_ref, b1_ref, w2_ref, b2_ref, w3_ref, b3_ref,
             out_ref):
        seg_ids = lax.broadcasted_iota(jnp.int32, (G, N), 0)
        onehot = (seg_ids == jnp.broadcast_to(b_ref[...], (G, N))
                  ).astype(jnp.float32)
        pooled = jnp.dot(onehot, h_ref[...], preferred_element_type=jnp.float32)
        z = jnp.maximum(
            jnp.dot(pooled, w1_ref[...], preferred_element_type=jnp.float32) + b1_ref[...], 0.0)
        z = jnp.maximum(
            jnp.dot(z, w2_ref[...], preferred_element_type=jnp.float32) + b2_ref[...], 0.0)
        logits = jnp.dot(z, w3_ref[...],
                         preferred_element_type=jnp.float32) + b3_ref[...]
        m = jnp.max(logits, axis=1, keepdims=True)
        e = jnp.exp(logits - m)
        lse = jnp.log(jnp.sum(e, axis=1, keepdims=True)) + m
        out_ref[...] = logits - lse

    return pl.pallas_call(
        body,
        out_shape=jax.ShapeDtypeStruct((G, C), jnp.float32),
    )(h, batch2d, fc1W, fc1b, fc2W, fc2b, fc3W, fc3b)


def kernel(x, edge_index, batch, params):
    src = edge_index[0].astype(jnp.int32).reshape(NW, EPT)
    dst = edge_index[1].astype(jnp.int32).reshape(NW, EPT)
    pad = EPAD - EPT
    srcp = jnp.concatenate(
        [src, jnp.zeros((NW, pad), jnp.int32)], axis=1).reshape(NW, CH, 1, K)
    dstp = jnp.concatenate(
        [dst, jnp.full((NW, pad), N, jnp.int32)], axis=1).reshape(NW, CH, 1, K)
    pidx = jnp.concatenate(
        [srcp, dstp], axis=2).reshape(NW, NG, GS, 2, K)
    zeros = jnp.zeros((AGG_ROWS, H), jnp.float32)
    batch2d = batch.astype(jnp.int32).reshape(1, N)

    h = x
    for p in params["convs"]:
        agg2 = _sc_agg(h, pidx, zeros)
        scale = (1.0 + p["eps"]).astype(jnp.float32).reshape(1, 1)
        h = _tc_layer(h, agg2, scale,
                      p["W1"], p["b1"].reshape(1, H),
                      p["W2"], p["b2"].reshape(1, H),
                      p["gamma"].reshape(1, H), p["beta"].reshape(1, H))

    return _tc_head(h, batch2d,
                    params["fc1W"], params["fc1b"].reshape(1, H),
                    params["fc2W"], params["fc2b"].reshape(1, H // 2),
                    params["fc3W"], params["fc3b"].reshape(1, C))


# speedup vs baseline: 1.0481x; 1.0481x over previous
import functools

import jax
import jax.numpy as jnp
from jax import lax
from jax.experimental import pallas as pl
from jax.experimental.pallas import tpu as pltpu
from jax.experimental.pallas import tpu_sc as plsc

N = 10000
E = 320000
D = 128
H = 128
C = 10
G = 64

NC = 2
NS = 16
NW = NC * NS
EPT = E // NW
K = 128
KH = K // 2
CH = 80
EPAD = CH * K
GS = 4
NG = CH // GS
ROWS_PER_TILE = 632
AGG_ROWS = ROWS_PER_TILE * NS


def _sc_agg(h, pidx, zeros):
    mesh = plsc.VectorSubcoreMesh(core_axis_name="c", subcore_axis_name="s")

    @functools.partial(
        pl.kernel,
        mesh=mesh,
        out_type=jax.ShapeDtypeStruct((NC, AGG_ROWS, H), jnp.float32),
        scratch_types=[
            pltpu.VMEM((GS, 2, K), jnp.int32),
            pltpu.VMEM((GS, 2, K), jnp.int32),
            pltpu.VMEM((GS, 2, K), jnp.int32),
            pltpu.VMEM((K, H), jnp.float32),
            pltpu.VMEM((K, H), jnp.float32),
            pltpu.VMEM_SHARED((AGG_ROWS, H), jnp.float32),
            pltpu.SemaphoreType.DMA,
            pltpu.SemaphoreType.DMA,
            pltpu.SemaphoreType.DMA,
            pltpu.SemaphoreType.DMA,
            pltpu.SemaphoreType.DMA,
            pltpu.SemaphoreType.DMA,
            pltpu.SemaphoreType.DMA,
            pltpu.SemaphoreType.DMA,
            pltpu.SemaphoreType.DMA,
        ],
    )
    def k(h_hbm, pidx_hbm, z_hbm, out_hbm,
          ibuf0, ibuf1, ibuf2, rows0, rows1, agg_sh,
          isem0, isem1, isem2, gsem0a, gsem0b, gsem1a, gsem1b,
          ssem0, ssem1):
        c = lax.axis_index("c")
        s = lax.axis_index("s")
        wid = c * NS + s
        ibuf = (ibuf0, ibuf1, ibuf2)
        rows = (rows0, rows1)
        isem = (isem0, isem1, isem2)
        gsem = ((gsem0a, gsem0b), (gsem1a, gsem1b))
        ssem = (ssem0, ssem1)

        pltpu.sync_copy(z_hbm.at[pl.ds(s * ROWS_PER_TILE, ROWS_PER_TILE)],
                        agg_sh.at[pl.ds(s * ROWS_PER_TILE, ROWS_PER_TILE)])
        plsc.subcore_barrier()

        def i_start(g, gb):
            pltpu.async_copy(pidx_hbm.at[wid, g], ibuf[gb], isem[gb])

        def i_wait(gb):
            pltpu.make_async_copy(pidx_hbm.at[0, 0], ibuf[gb],
                                  isem[gb]).wait()

        def g_start(t, gb, b):
            pltpu.async_copy(h_hbm.at[ibuf[gb].at[t, 0, pl.ds(0, KH)]],
                             rows[b].at[pl.ds(0, KH)], gsem[b][0])
            pltpu.async_copy(h_hbm.at[ibuf[gb].at[t, 0, pl.ds(KH, KH)]],
                             rows[b].at[pl.ds(KH, KH)], gsem[b][1])

        def g_wait(b):
            pltpu.make_async_copy(h_hbm.at[pl.ds(0, KH)],
                                  rows[b].at[pl.ds(0, KH)],
                                  gsem[b][0]).wait()
            pltpu.make_async_copy(h_hbm.at[pl.ds(0, KH)],
                                  rows[b].at[pl.ds(KH, KH)],
                                  gsem[b][1]).wait()

        def s_start(t, gb, b):
            pltpu.async_copy(rows[b], agg_sh.at[ibuf[gb].at[t, 1]], ssem[b],
                             add=True)

        def s_wait(b):
            pltpu.make_async_copy(rows[b], agg_sh.at[pl.ds(0, K)],
                                  ssem[b]).wait()

        i_start(0, 0)
        i_start(1, 1)
        i_wait(0)
        g_start(0, 0, 0)
        g_start(1, 0, 1)
        g_wait(0)
        s_start(0, 0, 0)
        s_wait(0)
        g_start(2, 0, 0)
        g_wait(1)
        s_start(1, 0, 1)
        i_start(2, 2)
        s_wait(1)
        g_start(3, 0, 1)
        g_wait(0)
        s_start(2, 0, 0)
        i_wait(1)
        s_wait(0)
        g_start(0, 1, 0)
        g_wait(1)
        s_start(3, 0, 1)
        s_wait(1)
        g_start(1, 1, 1)
        g_wait(0)
        s_start(0, 1, 0)
        s_wait(0)
        g_start(2, 1, 0)
        g_wait(1)
        s_start(1, 1, 1)
        i_start(3, 0)
        s_wait(1)
        g_start(3, 1, 1)
        g_wait(0)
        s_start(2, 1, 0)

        def block(jj, carry):
            for cst in range(3):
                g = 3 * jj + 2 + cst
                gb = (2 + cst) % 3
                pgb = (1 + cst) % 3
                for t in range(GS):
                    b = t % 2
                    pt, p_gb = (GS - 1, pgb) if t == 0 else (t - 1, gb)
                    if t == 0:
                        i_wait(gb)
                    s_wait(b)
                    g_start(t, gb, b)
                    g_wait(1 - b)
                    s_start(pt, p_gb, 1 - b)
                    if t == 2:
                        @pl.when(g + 2 < NG)
                        def _():
                            i_start(g + 2, (1 + cst) % 3)
            return carry

        lax.fori_loop(0, (NG - 2) // 3, block, 0)

        g_wait(1)
        s_start(GS - 1, 1, 1)
        s_wait(0)
        s_wait(1)

        plsc.subcore_barrier()
        pltpu.sync_copy(agg_sh.at[pl.ds(s * ROWS_PER_TILE, ROWS_PER_TILE)],
                        out_hbm.at[c, pl.ds(s * ROWS_PER_TILE, ROWS_PER_TILE)])

    return k(h, pidx, zeros)


def _tc_layer(h, agg2, scale, W1, b1, W2, b2, gamma, beta):
    def body(h_ref, agg_ref, sc_ref, w1_ref, b1_ref, w2_ref, b2_ref,
             g_ref, be_ref, out_ref):
        z = (h_ref[...] * sc_ref[0, 0] + agg_ref[0, pl.ds(0, N)]
             + agg_ref[1, pl.ds(0, N)])
        z = jnp.maximum(
            jnp.dot(z, w1_ref[...],
                    preferred_element_type=jnp.float32) + b1_ref[...], 0.0)
        z = jnp.maximum(
            jnp.dot(z, w2_ref[...],
                    preferred_element_type=jnp.float32) + b2_ref[...], 0.0)
        mean = jnp.sum(z, axis=0, keepdims=True) * (1.0 / N)
        d = z - mean
        var = jnp.sum(d * d, axis=0, keepdims=True) * (1.0 / N)
        zn = d * (g_ref[...] * lax.rsqrt(var + 1e-5)) + be_ref[...]
        out_ref[...] = jnp.maximum(zn, 0.0)

    return pl.pallas_call(
        body,
        out_shape=jax.ShapeDtypeStruct((N, H), jnp.float32),
    )(h, agg2, scale, W1, b1, W2, b2, gamma, beta)


def _tc_head(h, batch2d, fc1W, fc1b, fc2W, fc2b, fc3W, fc3b):
    def body(h_ref, b_ref, w1_ref, b1_ref, w2_ref, b2_ref, w3_ref, b3_ref,
             out_ref):
        seg_ids = lax.broadcasted_iota(jnp.int32, (G, N), 0)
        onehot = (seg_ids == jnp.broadcast_to(b_ref[...], (G, N))
                  ).astype(jnp.float32)
        pooled = jnp.dot(onehot, h_ref[...],
                         preferred_element_type=jnp.float32)
        z = jnp.maximum(
            jnp.dot(pooled, w1_ref[...],
                    preferred_element_type=jnp.float32) + b1_ref[...], 0.0)
        z = jnp.maximum(
            jnp.dot(z, w2_ref[...],
                    preferred_element_type=jnp.float32) + b2_ref[...], 0.0)
        logits = jnp.dot(z, w3_ref[...],
                         preferred_element_type=jnp.float32) + b3_ref[...]
        m = jnp.max(logits, axis=1, keepdims=True)
        e = jnp.exp(logits - m)
        lse = jnp.log(jnp.sum(e, axis=1, keepdims=True)) + m
        out_ref[...] = logits - lse

    return pl.pallas_call(
        body,
        out_shape=jax.ShapeDtypeStruct((G, C), jnp.float32),
    )(h, batch2d, fc1W, fc1b, fc2W, fc2b, fc3W, fc3b)


def kernel(x, edge_index, batch, params):
    src = edge_index[0].astype(jnp.int32).reshape(NW, EPT)
    dst = edge_index[1].astype(jnp.int32).reshape(NW, EPT)
    pad = EPAD - EPT
    srcp = jnp.concatenate(
        [src, jnp.zeros((NW, pad), jnp.int32)], axis=1).reshape(NW, CH, 1, K)
    dstp = jnp.concatenate(
        [dst, jnp.full((NW, pad), N, jnp.int32)], axis=1).reshape(NW, CH, 1, K)
    pidx = jnp.concatenate(
        [srcp, dstp], axis=2).reshape(NW, NG, GS, 2, K)
    zeros = jnp.zeros((AGG_ROWS, H), jnp.float32)
    batch2d = batch.astype(jnp.int32).reshape(1, N)

    h = x
    for p in params["convs"]:
        agg2 = _sc_agg(h, pidx, zeros)
        scale = (1.0 + p["eps"]).astype(jnp.float32).reshape(1, 1)
        h = _tc_layer(h, agg2, scale,
                      p["W1"], p["b1"].reshape(1, H),
                      p["W2"], p["b2"].reshape(1, H),
                      p["gamma"].reshape(1, H), p["beta"].reshape(1, H))

    return _tc_head(h, batch2d,
                    params["fc1W"], params["fc1b"].reshape(1, H),
                    params["fc2W"], params["fc2b"].reshape(1, H // 2),
                    params["fc3W"], params["fc3b"].reshape(1, C))


# speedup vs baseline: 1.0496x; 1.0014x over previous
import functools

import jax
import jax.numpy as jnp
from jax import lax
from jax.experimental import pallas as pl
from jax.experimental.pallas import tpu as pltpu
from jax.experimental.pallas import tpu_sc as plsc

N = 10000
E = 320000
D = 128
H = 128
C = 10
G = 64

NC = 2
NS = 16
NW = NC * NS
EPT = E // NW
K = 128
KH = K // 4
CH = 80
EPAD = CH * K
GS = 4
NG = CH // GS
ROWS_PER_TILE = 632
AGG_ROWS = ROWS_PER_TILE * NS


def _sc_agg(h, pidx, zeros):
    mesh = plsc.VectorSubcoreMesh(core_axis_name="c", subcore_axis_name="s")

    @functools.partial(
        pl.kernel,
        mesh=mesh,
        out_type=jax.ShapeDtypeStruct((NC, AGG_ROWS, H), jnp.float32),
        scratch_types=[
            pltpu.VMEM((GS, 2, K), jnp.int32),
            pltpu.VMEM((GS, 2, K), jnp.int32),
            pltpu.VMEM((GS, 2, K), jnp.int32),
            pltpu.VMEM((K, H), jnp.float32),
            pltpu.VMEM((K, H), jnp.float32),
            pltpu.VMEM_SHARED((AGG_ROWS, H), jnp.float32),
        ] + [pltpu.SemaphoreType.DMA] * 13,
    )
    def k(h_hbm, pidx_hbm, z_hbm, out_hbm,
          ibuf0, ibuf1, ibuf2, rows0, rows1, agg_sh, *sems):
        c = lax.axis_index("c")
        s = lax.axis_index("s")
        wid = c * NS + s
        ibuf = (ibuf0, ibuf1, ibuf2)
        rows = (rows0, rows1)
        isem = sems[0:3]
        gsem = (sems[3:7], sems[7:11])
        ssem = sems[11:13]

        def i_start(g, gb):
            pltpu.async_copy(pidx_hbm.at[wid, g], ibuf[gb], isem[gb])

        def i_wait(gb):
            pltpu.make_async_copy(pidx_hbm.at[0, 0], ibuf[gb],
                                  isem[gb]).wait()

        def g_start(t, gb, b):
            for q in range(4):
                pltpu.async_copy(
                    h_hbm.at[ibuf[gb].at[t, 0, pl.ds(q * KH, KH)]],
                    rows[b].at[pl.ds(q * KH, KH)], gsem[b][q])

        def g_wait(b):
            for q in range(4):
                pltpu.make_async_copy(h_hbm.at[pl.ds(0, KH)],
                                      rows[b].at[pl.ds(q * KH, KH)],
                                      gsem[b][q]).wait()

        def s_start(t, gb, b):
            pltpu.async_copy(rows[b], agg_sh.at[ibuf[gb].at[t, 1]], ssem[b],
                             add=True)

        def s_wait(b):
            pltpu.make_async_copy(rows[b], agg_sh.at[pl.ds(0, K)],
                                  ssem[b]).wait()

        i_start(0, 0)
        i_start(1, 1)
        i_wait(0)
        g_start(0, 0, 0)
        g_start(1, 0, 1)
        pltpu.sync_copy(z_hbm.at[pl.ds(s * ROWS_PER_TILE, ROWS_PER_TILE)],
                        agg_sh.at[pl.ds(s * ROWS_PER_TILE, ROWS_PER_TILE)])
        plsc.subcore_barrier()
        g_wait(0)
        s_start(0, 0, 0)
        s_wait(0)
        g_start(2, 0, 0)
        g_wait(1)
        s_start(1, 0, 1)
        i_start(2, 2)
        s_wait(1)
        g_start(3, 0, 1)
        g_wait(0)
        s_start(2, 0, 0)
        i_wait(1)
        s_wait(0)
        g_start(0, 1, 0)
        g_wait(1)
        s_start(3, 0, 1)
        s_wait(1)
        g_start(1, 1, 1)
        g_wait(0)
        s_start(0, 1, 0)
        s_wait(0)
        g_start(2, 1, 0)
        g_wait(1)
        s_start(1, 1, 1)
        i_start(3, 0)
        s_wait(1)
        g_start(3, 1, 1)
        g_wait(0)
        s_start(2, 1, 0)

        def block(jj, carry):
            for cst in range(3):
                g = 3 * jj + 2 + cst
                gb = (2 + cst) % 3
                pgb = (1 + cst) % 3
                for t in range(GS):
                    b = t % 2
                    pt, p_gb = (GS - 1, pgb) if t == 0 else (t - 1, gb)
                    if t == 0:
                        i_wait(gb)
                    s_wait(b)
                    g_start(t, gb, b)
                    g_wait(1 - b)
                    s_start(pt, p_gb, 1 - b)
                    if t == 2:
                        @pl.when(g + 2 < NG)
                        def _():
                            i_start(g + 2, (1 + cst) % 3)
            return carry

        lax.fori_loop(0, (NG - 2) // 3, block, 0)

        g_wait(1)
        s_start(GS - 1, 1, 1)
        s_wait(0)
        s_wait(1)

        plsc.subcore_barrier()
        pltpu.sync_copy(agg_sh.at[pl.ds(s * ROWS_PER_TILE, ROWS_PER_TILE)],
                        out_hbm.at[c, pl.ds(s * ROWS_PER_TILE, ROWS_PER_TILE)])

    return k(h, pidx, zeros)


def _tc_layer(h, agg2, scale, W1, b1, W2, b2, gamma, beta):
    def body(h_ref, agg_ref, sc_ref, w1_ref, b1_ref, w2_ref, b2_ref,
             g_ref, be_ref, out_ref):
        z = (h_ref[...] * sc_ref[0, 0] + agg_ref[0, pl.ds(0, N)]
             + agg_ref[1, pl.ds(0, N)])
        z = jnp.maximum(
            jnp.dot(z, w1_ref[...],
                    preferred_element_type=jnp.float32) + b1_ref[...], 0.0)
        z = jnp.maximum(
            jnp.dot(z, w2_ref[...],
                    preferred_element_type=jnp.float32) + b2_ref[...], 0.0)
        mean = jnp.sum(z, axis=0, keepdims=True) * (1.0 / N)
        d = z - mean
        var = jnp.sum(d * d, axis=0, keepdims=True) * (1.0 / N)
        zn = d * (g_ref[...] * lax.rsqrt(var + 1e-5)) + be_ref[...]
        out_ref[...] = jnp.maximum(zn, 0.0)

    return pl.pallas_call(
        body,
        out_shape=jax.ShapeDtypeStruct((N, H), jnp.float32),
    )(h, agg2, scale, W1, b1, W2, b2, gamma, beta)


def _tc_head(h, batch2d, fc1W, fc1b, fc2W, fc2b, fc3W, fc3b):
    def body(h_ref, b_ref, w1_ref, b1_ref, w2_ref, b2_ref, w3_ref, b3_ref,
             out_ref):
        seg_ids = lax.broadcasted_iota(jnp.int32, (G, N), 0)
        onehot = (seg_ids == jnp.broadcast_to(b_ref[...], (G, N))
                  ).astype(jnp.float32)
        pooled = jnp.dot(onehot, h_ref[...],
                         preferred_element_type=jnp.float32)
        z = jnp.maximum(
            jnp.dot(pooled, w1_ref[...],
                    preferred_element_type=jnp.float32) + b1_ref[...], 0.0)
        z = jnp.maximum(
            jnp.dot(z, w2_ref[...],
                    preferred_element_type=jnp.float32) + b2_ref[...], 0.0)
        logits = jnp.dot(z, w3_ref[...],
                         preferred_element_type=jnp.float32) + b3_ref[...]
        m = jnp.max(logits, axis=1, keepdims=True)
        e = jnp.exp(logits - m)
        lse = jnp.log(jnp.sum(e, axis=1, keepdims=True)) + m
        out_ref[...] = logits - lse

    return pl.pallas_call(
        body,
        out_shape=jax.ShapeDtypeStruct((G, C), jnp.float32),
    )(h, batch2d, fc1W, fc1b, fc2W, fc2b, fc3W, fc3b)


def kernel(x, edge_index, batch, params):
    src = edge_index[0].astype(jnp.int32).reshape(NW, EPT)
    dst = edge_index[1].astype(jnp.int32).reshape(NW, EPT)
    pad = EPAD - EPT
    srcp = jnp.concatenate(
        [src, jnp.zeros((NW, pad), jnp.int32)], axis=1).reshape(NW, CH, 1, K)
    dstp = jnp.concatenate(
        [dst, jnp.full((NW, pad), N, jnp.int32)], axis=1).reshape(NW, CH, 1, K)
    pidx = jnp.concatenate(
        [srcp, dstp], axis=2).reshape(NW, NG, GS, 2, K)
    zeros = jnp.zeros((AGG_ROWS, H), jnp.float32)
    batch2d = batch.astype(jnp.int32).reshape(1, N)

    h = x
    for p in params["convs"]:
        agg2 = _sc_agg(h, pidx, zeros)
        scale = (1.0 + p["eps"]).astype(jnp.float32).reshape(1, 1)
        h = _tc_layer(h, agg2, scale,
                      p["W1"], p["b1"].reshape(1, H),
                      p["W2"], p["b2"].reshape(1, H),
                      p["gamma"].reshape(1, H), p["beta"].reshape(1, H))

    return _tc_head(h, batch2d,
                    params["fc1W"], params["fc1b"].reshape(1, H),
                    params["fc2W"], params["fc2b"].reshape(1, H // 2),
                    params["fc3W"], params["fc3b"].reshape(1, C))
